# baseline (speedup 1.0000x reference)
import jax
import jax.numpy as jnp
from jax import lax
from jax.experimental import pallas as pl
from jax.experimental.pallas import tpu as pltpu
from jax.experimental.pallas import tpu_sc as plsc

NC = 2
NS = 16
NW = NC * NS
SZ = 128
GPC = 16
K = SZ * GPC
NCHUNK = 25
EPW = K * NCHUNK
EPAD = NW * EPW
CPAD = 1408
NPADT = 48
SZREL = 80


def _make_count_body(cntp):
    cpt = cntp // NS

    def body(skey_hbm, zc_hbm, ones_hbm, out_hbm, sk_v, ones_v, cnt_sh):
        cid = lax.axis_index("c")
        sid = lax.axis_index("s")
        wid = cid * NS + sid
        pltpu.sync_copy(zc_hbm, cnt_sh.at[pl.ds(sid * cpt, cpt)])
        pltpu.sync_copy(ones_hbm, ones_v)
        plsc.subcore_barrier()

        def chunk(j, carry):
            row0 = wid * (NCHUNK * GPC) + j * GPC
            pltpu.sync_copy(skey_hbm.at[pl.ds(row0, GPC)], sk_v)

            def group(m, c2):
                off = pl.multiple_of(m * SZ, SZ)
                pltpu.sync_copy(ones_v.at[pl.ds(off, SZ)],
                                cnt_sh.at[sk_v.at[m, 0]], add=True)
                return c2

            lax.fori_loop(0, GPC, group, 0)
            return carry

        lax.fori_loop(0, NCHUNK, chunk, 0)
        plsc.subcore_barrier()
        pltpu.sync_copy(cnt_sh.at[pl.ds(sid * cpt, cpt)],
                        out_hbm.at[cid, pl.ds(sid * cpt, cpt)])

    return body


def _make_edge_body(npad, rn, h):
    rpt = npad // NS
    nrel = (rn * h) // (SZREL * 128)

    def body(table128_hbm, gkey_hbm, skey_hbm, dst_hbm, inv_hbm, zr_hbm,
             out_hbm, gk_v, sk_v, d_v, rows_v, s_v, buf_v, buf2_v, acc_sh,
             tab_hbm, gsem, ssem):
        cid = lax.axis_index("c")
        sid = lax.axis_index("s")
        wid = cid * NS + sid
        pltpu.sync_copy(zr_hbm, acc_sh.at[pl.ds(sid * rpt, rpt)])

        rowsout = SZREL * 128 // h
        rph = 128 // h

        def relayout(j, carry):
            b = j * NS + sid

            @pl.when(b < nrel)
            def _():
                pltpu.sync_copy(table128_hbm.at[pl.ds(b * SZREL, SZREL)],
                                buf_v)

                def rl(p, c2):
                    for q in range(rph):
                        buf2_v[p * rph + q] = buf_v[p, pl.ds(q * h, h)]
                    return c2

                lax.fori_loop(0, SZREL, rl, 0)
                pltpu.sync_copy(buf2_v,
                                tab_hbm.at[cid, pl.ds(b * rowsout, rowsout)])
            return carry

        lax.fori_loop(0, (nrel + NS - 1) // NS, relayout, 0)
        plsc.subcore_barrier()
        table_hbm = tab_hbm.at[cid]

        def chunk(j, carry):
            row0 = wid * (NCHUNK * GPC) + j * GPC
            pltpu.sync_copy(gkey_hbm.at[pl.ds(row0, GPC)], gk_v)
            pltpu.sync_copy(skey_hbm.at[pl.ds(row0, GPC)], sk_v)
            pltpu.sync_copy(dst_hbm.at[pl.ds(row0, GPC)], d_v)

            def group(m, c2):
                off = pl.multiple_of(m * SZ, SZ)
                cpg = pltpu.async_copy(
                    table_hbm.at[gk_v.at[m, 0]],
                    rows_v.at[pl.ds(off, SZ)], gsem)
                cps = pltpu.async_copy(
                    inv_hbm.at[sk_v.at[m, 0]],
                    s_v.at[pl.ds(off, SZ)], ssem)
                cpg.wait()
                cps.wait()

                def scale(t, c3):
                    base = pl.multiple_of(off + t * 16, 16)
                    sv = s_v[pl.ds(base, 16)]
                    for i in range(16):
                        rows_v[base + i] = rows_v[base + i] * sv[i]
                    return c3

                lax.fori_loop(0, SZ // 16, scale, 0)
                pltpu.sync_copy(rows_v.at[pl.ds(off, SZ)],
                                acc_sh.at[d_v.at[m, 0]], add=True)
                return c2

            lax.fori_loop(0, GPC, group, 0)
            return carry

        lax.fori_loop(0, NCHUNK, chunk, 0)
        plsc.subcore_barrier()
        pltpu.sync_copy(acc_sh.at[pl.ds(sid * rpt, rpt)],
                        out_hbm.at[cid, pl.ds(sid * rpt, rpt)])

    return body


def kernel(edge_index, edge_type, basis1, comp1, root1, bias1,
           basis2, comp2, root2, bias2):
    N, H = root1.shape
    R, NB = comp1.shape
    C = root2.shape[1]
    E = edge_type.shape[0]
    f32 = jnp.float32
    mesh = plsc.VectorSubcoreMesh(core_axis_name="c", subcore_axis_name="s")

    cntp = R * N + CPAD
    npad = N + NPADT
    pade = EPAD - E

    src = edge_index[0]
    dst = edge_index[1]
    et = edge_type
    i32 = jnp.int32

    def pad3d(key, fill):
        keyp = jnp.concatenate(
            [key, jnp.full((pade,), fill, i32)])
        return keyp.reshape(EPAD // SZ, 1, SZ)

    gkey1 = pad3d(et * N + src, 0)
    gkey2 = pad3d(src * R + et, 0)
    skey = pad3d(et * N + dst, R * N)
    dst3d = pad3d(dst, npad - 1)

    cpt = cntp // NS
    rpt = npad // NS
    z_cnt = jnp.zeros((cpt,), f32)
    z_acc = jnp.zeros((rpt, H), f32)
    ones_k = jnp.ones((K,), f32)

    cnt_call = pl.kernel(
        _make_count_body(cntp),
        out_type=jax.ShapeDtypeStruct((NC, cntp), f32),
        mesh=mesh,
        compiler_params=pltpu.CompilerParams(use_tc_tiling_on_sc=False),
        scratch_types=[
            pltpu.VMEM((GPC, 1, SZ), jnp.int32),
            pltpu.VMEM((K,), f32),
            pltpu.VMEM_SHARED((cntp,), f32),
        ],
    )
    cnt_p = cnt_call(skey, z_cnt, ones_k)

    def _inv_body(c_ref, o_ref):
        o_ref[...] = (1.0 / jnp.maximum(c_ref[0] + c_ref[1], 1.0))[None]

    cblk = cntp // 49
    inv_cnt = pl.pallas_call(
        _inv_body,
        grid=(49,),
        in_specs=[pl.BlockSpec((2, cblk), lambda i: (0, i))],
        out_specs=pl.BlockSpec((1, cblk), lambda i: (0, i)),
        out_shape=jax.ShapeDtypeStruct((1, cntp), f32),
    )(cnt_p).reshape(cntp)

    def _w1_body(c_ref, b_ref, o_ref):
        o_ref[...] = jnp.dot(c_ref[...], b_ref[...],
                             preferred_element_type=f32)

    nh = N * H
    wblk = nh // 25
    w1 = pl.pallas_call(
        _w1_body,
        grid=(25,),
        in_specs=[pl.BlockSpec((R, NB), lambda i: (0, 0)),
                  pl.BlockSpec((NB, wblk), lambda i: (0, i))],
        out_specs=pl.BlockSpec((R, wblk), lambda i: (0, i)),
        out_shape=jax.ShapeDtypeStruct((R, nh), f32),
    )(comp1, basis1.reshape(NB, nh))
    table1 = w1.reshape((R * N * H) // 128, 128)

    edge_call = pl.kernel(
        _make_edge_body(npad, R * N, H),
        out_type=jax.ShapeDtypeStruct((NC, npad, H), f32),
        mesh=mesh,
        compiler_params=pltpu.CompilerParams(use_tc_tiling_on_sc=False),
        scratch_types=[
            pltpu.VMEM((GPC, 1, SZ), jnp.int32),
            pltpu.VMEM((GPC, 1, SZ), jnp.int32),
            pltpu.VMEM((GPC, 1, SZ), jnp.int32),
            pltpu.VMEM((K, H), f32),
            pltpu.VMEM((K,), f32),
            pltpu.VMEM((SZREL, 128), f32),
            pltpu.VMEM((SZREL * 128 // H, H), f32),
            pltpu.VMEM_SHARED((npad, H), f32),
            pltpu.HBM((NC, R * N, H), f32),
            pltpu.SemaphoreType.DMA,
            pltpu.SemaphoreType.DMA,
        ],
    )
    acc1_p = edge_call(table1, gkey1, skey, dst3d, inv_cnt, z_acc)

    w2 = (comp2 @ basis2.reshape(NB, H * C)).reshape(R, H, C)
    w2s = w2.transpose(1, 0, 2).reshape(H, R * C)
    rb1 = root1 + bias1[None, :]

    def _x_xw_body(p_ref, rb_ref, w2_ref, x_ref, xw_ref):
        xb = jnp.maximum(p_ref[0] + p_ref[1] + rb_ref[...], 0.0)
        x_ref[...] = xb
        xw_ref[...] = jnp.dot(xb, w2_ref[...], preferred_element_type=f32)

    nblk = N // 25
    x, xw = pl.pallas_call(
        _x_xw_body,
        grid=(25,),
        in_specs=[pl.BlockSpec((2, nblk, H), lambda i: (0, i, 0)),
                  pl.BlockSpec((nblk, H), lambda i: (i, 0)),
                  pl.BlockSpec((H, R * C), lambda i: (0, 0))],
        out_specs=[pl.BlockSpec((nblk, H), lambda i: (i, 0)),
                   pl.BlockSpec((nblk, R * C), lambda i: (i, 0))],
        out_shape=[jax.ShapeDtypeStruct((N, H), f32),
                   jax.ShapeDtypeStruct((N, R * C), f32)],
    )(acc1_p, rb1, w2s)
    table2 = xw.reshape((N * R * C) // 128, 128)

    acc2_p = edge_call(table2, gkey2, skey, dst3d, inv_cnt, z_acc)

    def _out_body(p_ref, x_ref, r2_ref, b2_ref, o_ref):
        o = (p_ref[0] + p_ref[1] + b2_ref[...]
             + jnp.dot(x_ref[...], r2_ref[...], preferred_element_type=f32))
        m = jnp.max(o, axis=-1, keepdims=True)
        s = o - m
        o_ref[...] = s - jnp.log(jnp.sum(jnp.exp(s), axis=-1, keepdims=True))

    out = pl.pallas_call(
        _out_body,
        grid=(25,),
        in_specs=[pl.BlockSpec((2, nblk, C), lambda i: (0, i, 0)),
                  pl.BlockSpec((nblk, H), lambda i: (i, 0)),
                  pl.BlockSpec((H, C), lambda i: (0, 0)),
                  pl.BlockSpec((1, C), lambda i: (0, 0))],
        out_specs=pl.BlockSpec((nblk, C), lambda i: (i, 0)),
        out_shape=jax.ShapeDtypeStruct((N, C), f32),
    )(acc2_p, x, root2, bias2[None, :])
    return out


# speedup vs baseline: 1.1179x; 1.1179x over previous
import jax
import jax.numpy as jnp
from jax import lax
from jax.experimental import pallas as pl
from jax.experimental.pallas import tpu as pltpu
from jax.experimental.pallas import tpu_sc as plsc

NC = 2
NS = 16
NW = NC * NS
SZ = 128
GPC = 16
K = SZ * GPC
NCHUNK = 25
EPW = K * NCHUNK
EPAD = NW * EPW
CPAD = 1408
NPADT = 176
OBLK = 448


def _make_count_body(cntp):
    cpt = cntp // NS

    def body(skey_hbm, zc_hbm, ones_hbm, out_hbm, sk_v, ones_v, cnt_sh):
        cid = lax.axis_index("c")
        sid = lax.axis_index("s")
        wid = cid * NS + sid
        pltpu.sync_copy(zc_hbm, cnt_sh.at[pl.ds(sid * cpt, cpt)])
        pltpu.sync_copy(ones_hbm, ones_v)
        plsc.subcore_barrier()

        def chunk(j, carry):
            row0 = wid * (NCHUNK * GPC) + j * GPC
            pltpu.sync_copy(skey_hbm.at[pl.ds(row0, GPC)], sk_v)

            def group(m, c2):
                off = pl.multiple_of(m * SZ, SZ)
                pltpu.sync_copy(ones_v.at[pl.ds(off, SZ)],
                                cnt_sh.at[sk_v.at[m, 0]], add=True)
                return c2

            lax.fori_loop(0, GPC, group, 0)
            return carry

        lax.fori_loop(0, NCHUNK, chunk, 0)
        plsc.subcore_barrier()
        pltpu.sync_copy(cnt_sh.at[pl.ds(sid * cpt, cpt)],
                        out_hbm.at[cid, pl.ds(sid * cpt, cpt)])

    return body


def _make_edge_body(npad, rn, h):
    rpt = npad // NS
    orpt = rpt * h // 128
    rph = 128 // h

    def body(table_hbm, gkey_hbm, skey_hbm, dst_hbm, inv_hbm, zr_hbm,
             out_hbm, gk_v, sk_v, d_v, rows_v, s_v, ob_v, ob128_v, acc_sh,
             gsem, ssem):
        cid = lax.axis_index("c")
        sid = lax.axis_index("s")
        wid = cid * NS + sid
        pltpu.sync_copy(zr_hbm, acc_sh.at[pl.ds(sid * rpt, rpt)])
        plsc.subcore_barrier()

        def chunk(j, carry):
            row0 = wid * (NCHUNK * GPC) + j * GPC
            pltpu.sync_copy(gkey_hbm.at[pl.ds(row0, GPC)], gk_v)
            pltpu.sync_copy(skey_hbm.at[pl.ds(row0, GPC)], sk_v)
            pltpu.sync_copy(dst_hbm.at[pl.ds(row0, GPC)], d_v)

            def group(m, c2):
                off = pl.multiple_of(m * SZ, SZ)
                cpg = pltpu.async_copy(
                    table_hbm.at[gk_v.at[m, 0]],
                    rows_v.at[pl.ds(off, SZ)], gsem)
                cps = pltpu.async_copy(
                    inv_hbm.at[sk_v.at[m, 0]],
                    s_v.at[pl.ds(off, SZ)], ssem)
                cpg.wait()
                cps.wait()

                def scale(t, c3):
                    base = pl.multiple_of(off + t * 16, 16)
                    sv = s_v[pl.ds(base, 16)]
                    for i in range(16):
                        rows_v[base + i] = rows_v[base + i] * sv[i]
                    return c3

                lax.fori_loop(0, SZ // 16, scale, 0)
                pltpu.sync_copy(rows_v.at[pl.ds(off, SZ)],
                                acc_sh.at[d_v.at[m, 0]], add=True)
                return c2

            lax.fori_loop(0, GPC, group, 0)
            return carry

        lax.fori_loop(0, NCHUNK, chunk, 0)
        plsc.subcore_barrier()

        def outblk(t, carry):
            pltpu.sync_copy(
                acc_sh.at[pl.ds(sid * rpt + t * OBLK, OBLK)], ob_v)

            def merge(p, c2):
                for q in range(rph):
                    ob128_v[p, pl.ds(q * h, h)] = ob_v[p * rph + q]
                return c2

            lax.fori_loop(0, OBLK // rph, merge, 0)
            pltpu.sync_copy(
                ob128_v,
                out_hbm.at[cid, pl.ds(sid * orpt + t * (OBLK // rph),
                                      OBLK // rph)])
            return carry

        lax.fori_loop(0, rpt // OBLK, outblk, 0)

    return body


def kernel(edge_index, edge_type, basis1, comp1, root1, bias1,
           basis2, comp2, root2, bias2):
    N, H = root1.shape
    R, NB = comp1.shape
    C = root2.shape[1]
    E = edge_type.shape[0]
    f32 = jnp.float32
    mesh = plsc.VectorSubcoreMesh(core_axis_name="c", subcore_axis_name="s")

    cntp = R * N + CPAD
    npad = N + NPADT
    nacc = npad * H // 128
    pade = EPAD - E

    src = edge_index[0]
    dst = edge_index[1]
    et = edge_type
    i32 = jnp.int32

    def pad3d(key, fill):
        keyp = jnp.concatenate(
            [key, jnp.full((pade,), fill, i32)])
        return keyp.reshape(EPAD // SZ, 1, SZ)

    gkey1 = pad3d(et * N + src, 0)
    gkey2 = pad3d(src * R + et, 0)
    skey = pad3d(et * N + dst, R * N)
    dst3d = pad3d(dst, npad - 1)

    cpt = cntp // NS
    rpt = npad // NS
    z_cnt = jnp.zeros((cpt,), f32)
    z_acc = jnp.zeros((rpt, H), f32)
    ones_k = jnp.ones((K,), f32)

    cnt_call = pl.kernel(
        _make_count_body(cntp),
        out_type=jax.ShapeDtypeStruct((NC, cntp), f32),
        mesh=mesh,
        compiler_params=pltpu.CompilerParams(use_tc_tiling_on_sc=False),
        scratch_types=[
            pltpu.VMEM((GPC, 1, SZ), jnp.int32),
            pltpu.VMEM((K,), f32),
            pltpu.VMEM_SHARED((cntp,), f32),
        ],
    )
    cnt_p = cnt_call(skey, z_cnt, ones_k)

    def _inv_body(c_ref, o_ref):
        o_ref[...] = (1.0 / jnp.maximum(c_ref[0] + c_ref[1], 1.0))[None]

    cblk = cntp // 49
    inv_cnt = pl.pallas_call(
        _inv_body,
        grid=(49,),
        in_specs=[pl.BlockSpec((2, cblk), lambda i: (0, i))],
        out_specs=pl.BlockSpec((1, cblk), lambda i: (0, i)),
        out_shape=jax.ShapeDtypeStruct((1, cntp), f32),
    )(cnt_p).reshape(cntp)

    def _w1_body(c_ref, b_ref, o_ref):
        o_ref[...] = jnp.dot(c_ref[...], b_ref[...],
                             preferred_element_type=f32)

    nh = N * H
    wblk = nh // 25
    w1 = pl.pallas_call(
        _w1_body,
        grid=(25,),
        in_specs=[pl.BlockSpec((R, NB), lambda i: (0, 0)),
                  pl.BlockSpec((NB, wblk), lambda i: (0, i))],
        out_specs=pl.BlockSpec((R, wblk), lambda i: (0, i)),
        out_shape=jax.ShapeDtypeStruct((R, nh), f32),
    )(comp1, basis1.reshape(NB, nh))
    table1 = w1.reshape(R * N, H)

    def make_edge_call(rn):
        return pl.kernel(
            _make_edge_body(npad, rn, H),
            out_type=jax.ShapeDtypeStruct((NC, nacc, 128), f32),
            mesh=mesh,
            compiler_params=pltpu.CompilerParams(use_tc_tiling_on_sc=False),
            scratch_types=[
                pltpu.VMEM((GPC, 1, SZ), jnp.int32),
                pltpu.VMEM((GPC, 1, SZ), jnp.int32),
                pltpu.VMEM((GPC, 1, SZ), jnp.int32),
                pltpu.VMEM((K, H), f32),
                pltpu.VMEM((K,), f32),
                pltpu.VMEM((OBLK, H), f32),
                pltpu.VMEM((OBLK * H // 128, 128), f32),
                pltpu.VMEM_SHARED((npad, H), f32),
                pltpu.SemaphoreType.DMA,
                pltpu.SemaphoreType.DMA,
            ],
        )

    acc1_p = make_edge_call(R * N)(table1, gkey1, skey, dst3d, inv_cnt, z_acc)

    w2 = (comp2 @ basis2.reshape(NB, H * C)).reshape(R, H, C)
    w2s = w2.transpose(1, 0, 2).reshape(H, R * C)
    eye8 = jnp.eye(128 // H, dtype=f32)
    w2big = jnp.kron(eye8, w2s)
    r2big = jnp.kron(eye8, root2)
    rb1 = root1 + bias1[None, :]
    rb1p = jnp.concatenate([rb1, jnp.zeros((NPADT, H), f32)])
    rb128 = rb1p.reshape(nacc, 128)

    nb128 = nacc // 16

    def _x_xw_body(p_ref, rb_ref, w2_ref, x_ref, xw_ref):
        xb = jnp.maximum(p_ref[0] + p_ref[1] + rb_ref[...], 0.0)
        x_ref[...] = xb
        xw_ref[...] = jnp.dot(xb, w2_ref[...], preferred_element_type=f32)

    x128, xw = pl.pallas_call(
        _x_xw_body,
        grid=(16,),
        in_specs=[pl.BlockSpec((2, nb128, 128), lambda i: (0, i, 0)),
                  pl.BlockSpec((nb128, 128), lambda i: (i, 0)),
                  pl.BlockSpec((128, (128 // H) * R * C), lambda i: (0, 0))],
        out_specs=[pl.BlockSpec((nb128, 128), lambda i: (i, 0)),
                   pl.BlockSpec((nb128, (128 // H) * R * C),
                                lambda i: (i, 0))],
        out_shape=[jax.ShapeDtypeStruct((nacc, 128), f32),
                   jax.ShapeDtypeStruct((nacc, (128 // H) * R * C), f32)],
    )(acc1_p, rb128, w2big)
    table2 = xw.reshape(npad * R, C)

    acc2_p = make_edge_call(npad * R)(table2, gkey2, skey, dst3d, inv_cnt,
                                      z_acc)

    gsz = 128 // C
    onesb = jnp.kron(eye8, jnp.ones((C, C), f32))
    lead = jnp.kron(eye8, jnp.zeros((C, C), f32).at[0].set(1.0))
    b2t = jnp.tile(bias2, gsz)[None, :]

    def _out_body(p_ref, x_ref, r2_ref, b2_ref, ones_ref, lead_ref, o_ref):
        o = (p_ref[0] + p_ref[1] + b2_ref[...]
             + jnp.dot(x_ref[...], r2_ref[...], preferred_element_type=f32))
        m = o
        for k in (1, 2, 4, 8):
            m = jnp.maximum(m, pltpu.roll(m, 128 - k, axis=1))
        mb = jnp.dot(m, lead_ref[...], preferred_element_type=f32)
        s = o - mb
        e = jnp.exp(s)
        lse = jnp.log(jnp.dot(e, ones_ref[...], preferred_element_type=f32))
        o_ref[...] = s - lse

    out128 = pl.pallas_call(
        _out_body,
        grid=(16,),
        in_specs=[pl.BlockSpec((2, nb128, 128), lambda i: (0, i, 0)),
                  pl.BlockSpec((nb128, 128), lambda i: (i, 0)),
                  pl.BlockSpec((128, 128), lambda i: (0, 0)),
                  pl.BlockSpec((1, 128), lambda i: (0, 0)),
                  pl.BlockSpec((128, 128), lambda i: (0, 0)),
                  pl.BlockSpec((128, 128), lambda i: (0, 0))],
        out_specs=pl.BlockSpec((nb128, 128), lambda i: (i, 0)),
        out_shape=jax.ShapeDtypeStruct((nacc, 128), f32),
    )(acc2_p, x128, r2big, b2t, onesb, lead)
    return out128.reshape(npad, C)[:N]


# speedup vs baseline: 1.2422x; 1.1112x over previous
import jax
import jax.numpy as jnp
from jax import lax
from jax.experimental import pallas as pl
from jax.experimental.pallas import tpu as pltpu
from jax.experimental.pallas import tpu_sc as plsc

NC = 2
NS = 16
NW = NC * NS
SZ = 128
GPC = 16
K = SZ * GPC
NCHUNK = 25
EPW = K * NCHUNK
EPAD = NW * EPW
CPAD = 1408
NPADT = 176
OBLK = 448


def _make_count_body(cntp):
    cpt = cntp // NS

    def body(skey_hbm, zc_hbm, ones_hbm, out_hbm, sk_v, ones_v, cnt_sh):
        cid = lax.axis_index("c")
        sid = lax.axis_index("s")
        wid = cid * NS + sid
        pltpu.sync_copy(zc_hbm, cnt_sh.at[pl.ds(sid * cpt, cpt)])
        pltpu.sync_copy(ones_hbm, ones_v)
        plsc.subcore_barrier()

        def chunk(j, carry):
            row0 = wid * (NCHUNK * GPC) + j * GPC
            pltpu.sync_copy(skey_hbm.at[pl.ds(row0, GPC)], sk_v)

            def group(m, c2):
                off = pl.multiple_of(m * SZ, SZ)
                pltpu.sync_copy(ones_v.at[pl.ds(off, SZ)],
                                cnt_sh.at[sk_v.at[m, 0]], add=True)
                return c2

            lax.fori_loop(0, GPC, group, 0)
            return carry

        lax.fori_loop(0, NCHUNK, chunk, 0)
        plsc.subcore_barrier()
        pltpu.sync_copy(cnt_sh.at[pl.ds(sid * cpt, cpt)],
                        out_hbm.at[cid, pl.ds(sid * cpt, cpt)])

    return body


def _make_edge_body(npad, rn, h):
    rpt = npad // NS
    orpt = rpt * h // 128
    rph = 128 // h

    def body(table_hbm, gkey_hbm, skey_hbm, dst_hbm, inv_hbm, zr_hbm,
             out_hbm, gk_v, sk_v, d_v, rows_v, s_v, ob_v, ob128_v, acc_sh,
             gsem, ssem):
        cid = lax.axis_index("c")
        sid = lax.axis_index("s")
        wid = cid * NS + sid
        pltpu.sync_copy(zr_hbm, acc_sh.at[pl.ds(sid * rpt, rpt)])
        plsc.subcore_barrier()

        def chunk(j, carry):
            row0 = wid * (NCHUNK * GPC) + j * GPC
            pltpu.sync_copy(gkey_hbm.at[pl.ds(row0, GPC)], gk_v)
            pltpu.sync_copy(skey_hbm.at[pl.ds(row0, GPC)], sk_v)
            pltpu.sync_copy(dst_hbm.at[pl.ds(row0, GPC)], d_v)

            def group(m, c2):
                off = pl.multiple_of(m * SZ, SZ)
                cpg = pltpu.async_copy(
                    table_hbm.at[gk_v.at[m, 0]],
                    rows_v.at[pl.ds(off, SZ)], gsem)
                cps = pltpu.async_copy(
                    inv_hbm.at[sk_v.at[m, 0]],
                    s_v.at[pl.ds(off, SZ)], ssem)
                cpg.wait()
                cps.wait()

                def scale(t, c3):
                    base = pl.multiple_of(off + t * 16, 16)
                    sv = s_v[pl.ds(base, 16)]
                    for i in range(16):
                        rows_v[base + i] = rows_v[base + i] * sv[i]
                    return c3

                lax.fori_loop(0, SZ // 16, scale, 0)
                pltpu.sync_copy(rows_v.at[pl.ds(off, SZ)],
                                acc_sh.at[d_v.at[m, 0]], add=True)
                return c2

            lax.fori_loop(0, GPC, group, 0)
            return carry

        lax.fori_loop(0, NCHUNK, chunk, 0)
        plsc.subcore_barrier()

        def outblk(t, carry):
            pltpu.sync_copy(
                acc_sh.at[pl.ds(sid * rpt + t * OBLK, OBLK)], ob_v)

            def merge(p, c2):
                for q in range(rph):
                    ob128_v[p, pl.ds(q * h, h)] = ob_v[p * rph + q]
                return c2

            lax.fori_loop(0, OBLK // rph, merge, 0)
            pltpu.sync_copy(
                ob128_v,
                out_hbm.at[cid, pl.ds(sid * orpt + t * (OBLK // rph),
                                      OBLK // rph)])
            return carry

        lax.fori_loop(0, rpt // OBLK, outblk, 0)

    return body


def kernel(edge_index, edge_type, basis1, comp1, root1, bias1,
           basis2, comp2, root2, bias2):
    N, H = root1.shape
    R, NB = comp1.shape
    C = root2.shape[1]
    E = edge_type.shape[0]
    f32 = jnp.float32
    mesh = plsc.VectorSubcoreMesh(core_axis_name="c", subcore_axis_name="s")

    cntp = R * N + CPAD
    npad = N + NPADT
    nacc = npad * H // 128
    pade = EPAD - E

    src = edge_index[0]
    dst = edge_index[1]
    et = edge_type
    i32 = jnp.int32

    def pad3d(key, fill):
        keyp = jnp.concatenate(
            [key, jnp.full((pade,), fill, i32)])
        return keyp.reshape(EPAD // SZ, 1, SZ)

    gkey1 = pad3d(et * N + src, 0)
    gkey2 = pad3d(src * R + et, 0)
    skey = pad3d(et * N + dst, R * N)
    dst3d = pad3d(dst, npad - 1)

    cpt = cntp // NS
    rpt = npad // NS
    z_cnt = jnp.zeros((cpt,), f32)
    z_acc = jnp.zeros((rpt, H), f32)
    ones_k = jnp.ones((K,), f32)

    cnt_call = pl.kernel(
        _make_count_body(cntp),
        out_type=jax.ShapeDtypeStruct((NC, cntp), f32),
        mesh=mesh,
        compiler_params=pltpu.CompilerParams(use_tc_tiling_on_sc=False),
        scratch_types=[
            pltpu.VMEM((GPC, 1, SZ), jnp.int32),
            pltpu.VMEM((K,), f32),
            pltpu.VMEM_SHARED((cntp,), f32),
        ],
    )
    cnt_p = cnt_call(skey, z_cnt, ones_k)

    def _inv_body(c_ref, o_ref):
        o_ref[...] = (1.0 / jnp.maximum(c_ref[0] + c_ref[1], 1.0))[None]

    cblk = cntp // 49
    inv_cnt = pl.pallas_call(
        _inv_body,
        grid=(49,),
        in_specs=[pl.BlockSpec((2, cblk), lambda i: (0, i))],
        out_specs=pl.BlockSpec((1, cblk), lambda i: (0, i)),
        out_shape=jax.ShapeDtypeStruct((1, cntp), f32),
    )(cnt_p).reshape(cntp)

    basisT2d = basis1.transpose(0, 2, 1).reshape(NB, H * N)
    wblk = (H * N) // 25

    def _w1_body(c_ref, b_ref, o_ref):
        o_ref[...] = jnp.dot(c_ref[...], b_ref[...],
                             preferred_element_type=f32)

    w1hn = pl.pallas_call(
        _w1_body,
        grid=(25,),
        in_specs=[pl.BlockSpec((R, NB), lambda i: (0, 0)),
                  pl.BlockSpec((NB, wblk), lambda i: (0, i))],
        out_specs=pl.BlockSpec((R, wblk), lambda i: (0, i)),
        out_shape=jax.ShapeDtypeStruct((R, H * N), f32),
    )(comp1, basisT2d)

    eye16 = jnp.eye(H, dtype=f32)

    def _tr_body(a_ref, e_ref, o_ref):
        o_ref[0] = lax.dot_general(a_ref[0], e_ref[...],
                                   (((0,), (0,)), ((), ())),
                                   preferred_element_type=f32)

    w1 = pl.pallas_call(
        _tr_body,
        grid=(R,),
        in_specs=[pl.BlockSpec((1, H, N), lambda i: (i, 0, 0)),
                  pl.BlockSpec((H, H), lambda i: (0, 0))],
        out_specs=pl.BlockSpec((1, N, H), lambda i: (i, 0, 0)),
        out_shape=jax.ShapeDtypeStruct((R, N, H), f32),
    )(w1hn.reshape(R, H, N), eye16)
    table1 = w1.reshape(R * N, H)

    def make_edge_call(rn):
        return pl.kernel(
            _make_edge_body(npad, rn, H),
            out_type=jax.ShapeDtypeStruct((NC, nacc, 128), f32),
            mesh=mesh,
            compiler_params=pltpu.CompilerParams(use_tc_tiling_on_sc=False),
            scratch_types=[
                pltpu.VMEM((GPC, 1, SZ), jnp.int32),
                pltpu.VMEM((GPC, 1, SZ), jnp.int32),
                pltpu.VMEM((GPC, 1, SZ), jnp.int32),
                pltpu.VMEM((K, H), f32),
                pltpu.VMEM((K,), f32),
                pltpu.VMEM((OBLK, H), f32),
                pltpu.VMEM((OBLK * H // 128, 128), f32),
                pltpu.VMEM_SHARED((npad, H), f32),
                pltpu.SemaphoreType.DMA,
                pltpu.SemaphoreType.DMA,
            ],
        )

    acc1_p = make_edge_call(R * N)(table1, gkey1, skey, dst3d, inv_cnt, z_acc)

    w2 = (comp2 @ basis2.reshape(NB, H * C)).reshape(R, H, C)
    w2s = w2.transpose(1, 0, 2).reshape(H, R * C)
    eye8 = jnp.eye(128 // H, dtype=f32)
    w2big = jnp.kron(eye8, w2s)
    r2big = jnp.kron(eye8, root2)
    rb1 = root1 + bias1[None, :]
    rb1p = jnp.concatenate([rb1, jnp.zeros((NPADT, H), f32)])
    rb128 = rb1p.reshape(nacc, 128)

    nb128 = nacc // 16

    def _x_xw_body(p_ref, rb_ref, w2_ref, x_ref, xw_ref):
        xb = jnp.maximum(p_ref[0] + p_ref[1] + rb_ref[...], 0.0)
        x_ref[...] = xb
        xw_ref[...] = jnp.dot(xb, w2_ref[...], preferred_element_type=f32)

    x128, xw = pl.pallas_call(
        _x_xw_body,
        grid=(16,),
        in_specs=[pl.BlockSpec((2, nb128, 128), lambda i: (0, i, 0)),
                  pl.BlockSpec((nb128, 128), lambda i: (i, 0)),
                  pl.BlockSpec((128, (128 // H) * R * C), lambda i: (0, 0))],
        out_specs=[pl.BlockSpec((nb128, 128), lambda i: (i, 0)),
                   pl.BlockSpec((nb128, (128 // H) * R * C),
                                lambda i: (i, 0))],
        out_shape=[jax.ShapeDtypeStruct((nacc, 128), f32),
                   jax.ShapeDtypeStruct((nacc, (128 // H) * R * C), f32)],
    )(acc1_p, rb128, w2big)
    table2 = xw.reshape(npad * R, C)

    acc2_p = make_edge_call(npad * R)(table2, gkey2, skey, dst3d, inv_cnt,
                                      z_acc)

    gsz = 128 // C
    onesb = jnp.kron(eye8, jnp.ones((C, C), f32))
    lead = jnp.kron(eye8, jnp.zeros((C, C), f32).at[0].set(1.0))
    b2t = jnp.tile(bias2, gsz)[None, :]

    def _out_body(p_ref, x_ref, r2_ref, b2_ref, ones_ref, lead_ref, o_ref):
        o = (p_ref[0] + p_ref[1] + b2_ref[...]
             + jnp.dot(x_ref[...], r2_ref[...], preferred_element_type=f32))
        m = o
        for k in (1, 2, 4, 8):
            m = jnp.maximum(m, pltpu.roll(m, 128 - k, axis=1))
        mb = jnp.dot(m, lead_ref[...], preferred_element_type=f32)
        s = o - mb
        e = jnp.exp(s)
        lse = jnp.log(jnp.dot(e, ones_ref[...], preferred_element_type=f32))
        o_ref[...] = s - lse

    out128 = pl.pallas_call(
        _out_body,
        grid=(16,),
        in_specs=[pl.BlockSpec((2, nb128, 128), lambda i: (0, i, 0)),
                  pl.BlockSpec((nb128, 128), lambda i: (i, 0)),
                  pl.BlockSpec((128, 128), lambda i: (0, 0)),
                  pl.BlockSpec((1, 128), lambda i: (0, 0)),
                  pl.BlockSpec((128, 128), lambda i: (0, 0)),
                  pl.BlockSpec((128, 128), lambda i: (0, 0))],
        out_specs=pl.BlockSpec((nb128, 128), lambda i: (i, 0)),
        out_shape=jax.ShapeDtypeStruct((nacc, 128), f32),
    )(acc2_p, x128, r2big, b2t, onesb, lead)
    return out128.reshape(npad, C)[:N]


# speedup vs baseline: 1.5618x; 1.2572x over previous
import jax
import jax.numpy as jnp
from jax import lax
from jax.experimental import pallas as pl
from jax.experimental.pallas import tpu as pltpu
from jax.experimental.pallas import tpu_sc as plsc

NC = 2
NS = 16
NW = NC * NS
SZ = 128
GPC = 16
K = SZ * GPC
NCHUNK = 25
EPW = K * NCHUNK
EPAD = NW * EPW
CPAD = 1408
NPADT = 176
OBLK = 448


def _make_count_body(cntp):
    cpt = cntp // NS

    def body(skey_hbm, zc_hbm, ones_hbm, out_hbm, sk_v, ones_v, cnt_sh):
        cid = lax.axis_index("c")
        sid = lax.axis_index("s")
        wid = cid * NS + sid
        pltpu.sync_copy(zc_hbm, cnt_sh.at[pl.ds(sid * cpt, cpt)])
        pltpu.sync_copy(ones_hbm, ones_v)
        plsc.subcore_barrier()

        def chunk(j, carry):
            row0 = wid * (NCHUNK * GPC) + j * GPC
            pltpu.sync_copy(skey_hbm.at[pl.ds(row0, GPC)], sk_v)

            def group(m, c2):
                off = pl.multiple_of(m * SZ, SZ)
                pltpu.sync_copy(ones_v.at[pl.ds(off, SZ)],
                                cnt_sh.at[sk_v.at[m, 0]], add=True)
                return c2

            lax.fori_loop(0, GPC, group, 0)
            return carry

        lax.fori_loop(0, NCHUNK, chunk, 0)
        plsc.subcore_barrier()
        pltpu.sync_copy(cnt_sh.at[pl.ds(sid * cpt, cpt)],
                        out_hbm.at[cid, pl.ds(sid * cpt, cpt)])

    return body


def _make_edge_body(npad, rn, h):
    rpt = npad // NS
    orpt = rpt * h // 128
    rph = 128 // h

    def body(table_hbm, gkey_hbm, skey_hbm, dst_hbm, inv_hbm, zr_hbm,
             out_hbm, gk_v, sk_v, d_v, rows_v, s_v, ob_v, ob128_v, acc_sh,
             gsa, gsb, ssa, ssb, isem, scsem):
        cid = lax.axis_index("c")
        sid = lax.axis_index("s")
        wid = cid * NS + sid
        pltpu.sync_copy(zr_hbm, acc_sh.at[pl.ds(sid * rpt, rpt)])
        plsc.subcore_barrier()

        def idx_issue(j, jp):
            row0 = wid * (NCHUNK * GPC) + j * GPC
            pltpu.async_copy(gkey_hbm.at[pl.ds(row0, GPC)], gk_v.at[jp],
                             isem)
            pltpu.async_copy(skey_hbm.at[pl.ds(row0, GPC)], sk_v.at[jp],
                             isem)
            pltpu.async_copy(dst_hbm.at[pl.ds(row0, GPC)], d_v.at[jp], isem)

        def idx_wait(jp):
            row0 = wid * (NCHUNK * GPC)
            pltpu.make_async_copy(gkey_hbm.at[pl.ds(row0, GPC)],
                                  gk_v.at[jp], isem).wait()
            pltpu.make_async_copy(skey_hbm.at[pl.ds(row0, GPC)],
                                  sk_v.at[jp], isem).wait()
            pltpu.make_async_copy(dst_hbm.at[pl.ds(row0, GPC)],
                                  d_v.at[jp], isem).wait()

        def gath_issue(jp, m, gs, ss):
            off = pl.multiple_of(m * SZ, SZ)
            pltpu.async_copy(table_hbm.at[gk_v.at[jp, m, 0]],
                             rows_v.at[pl.ds(off, SZ)], gs)
            pltpu.async_copy(inv_hbm.at[sk_v.at[jp, m, 0]],
                             s_v.at[pl.ds(off, SZ)], ss)

        def gath_wait(jp, m, gs, ss):
            off = pl.multiple_of(m * SZ, SZ)
            pltpu.make_async_copy(table_hbm.at[gk_v.at[jp, m, 0]],
                                  rows_v.at[pl.ds(off, SZ)], gs).wait()
            pltpu.make_async_copy(inv_hbm.at[sk_v.at[jp, m, 0]],
                                  s_v.at[pl.ds(off, SZ)], ss).wait()

        def process(jp, m):
            off = pl.multiple_of(m * SZ, SZ)

            def scale(t, c3):
                base = pl.multiple_of(off + t * 16, 16)
                sv = s_v[pl.ds(base, 16)]
                for i in range(16):
                    rows_v[base + i] = rows_v[base + i] * sv[i]
                return c3

            lax.fori_loop(0, SZ // 16, scale, 0)
            pltpu.async_copy(rows_v.at[pl.ds(off, SZ)],
                             acc_sh.at[d_v.at[jp, m, 0]], scsem, add=True)

        def sc_drain(jp):
            def one(m, c2):
                off = pl.multiple_of(m * SZ, SZ)
                pltpu.make_async_copy(rows_v.at[pl.ds(off, SZ)],
                                      acc_sh.at[d_v.at[jp, m, 0]],
                                      scsem).wait()
                return c2

            lax.fori_loop(0, GPC, one, 0)

        idx_issue(0, 0)

        def chunk(j, carry):
            jp = lax.rem(j, 2)
            idx_wait(jp)

            @pl.when(j + 1 < NCHUNK)
            def _():
                idx_issue(j + 1, 1 - jp)

            gath_issue(jp, 0, gsa, ssa)

            def pair(m2, c2):
                m = m2 * 2
                gath_issue(jp, m + 1, gsb, ssb)
                gath_wait(jp, m, gsa, ssa)
                process(jp, m)

                @pl.when(m2 + 1 < GPC // 2)
                def _():
                    gath_issue(jp, m + 2, gsa, ssa)

                gath_wait(jp, m + 1, gsb, ssb)
                process(jp, m + 1)
                return c2

            lax.fori_loop(0, GPC // 2, pair, 0)
            sc_drain(jp)
            return carry

        lax.fori_loop(0, NCHUNK, chunk, 0)
        plsc.subcore_barrier()

        def outblk(t, carry):
            pltpu.sync_copy(
                acc_sh.at[pl.ds(sid * rpt + t * OBLK, OBLK)], ob_v)

            def merge(p, c2):
                for q in range(rph):
                    ob128_v[p, pl.ds(q * h, h)] = ob_v[p * rph + q]
                return c2

            lax.fori_loop(0, OBLK // rph, merge, 0)
            pltpu.sync_copy(
                ob128_v,
                out_hbm.at[cid, pl.ds(sid * orpt + t * (OBLK // rph),
                                      OBLK // rph)])
            return carry

        lax.fori_loop(0, rpt // OBLK, outblk, 0)

    return body


def kernel(edge_index, edge_type, basis1, comp1, root1, bias1,
           basis2, comp2, root2, bias2):
    N, H = root1.shape
    R, NB = comp1.shape
    C = root2.shape[1]
    E = edge_type.shape[0]
    f32 = jnp.float32
    mesh = plsc.VectorSubcoreMesh(core_axis_name="c", subcore_axis_name="s")

    cntp = R * N + CPAD
    npad = N + NPADT
    nacc = npad * H // 128
    pade = EPAD - E

    src = edge_index[0]
    dst = edge_index[1]
    et = edge_type
    i32 = jnp.int32

    def pad3d(key, fill):
        keyp = jnp.concatenate(
            [key, jnp.full((pade,), fill, i32)])
        return keyp.reshape(EPAD // SZ, 1, SZ)

    gkey1 = pad3d(et * N + src, 0)
    gkey2 = pad3d(src * R + et, 0)
    skey = pad3d(et * N + dst, R * N)
    dst3d = pad3d(dst, npad - 1)

    cpt = cntp // NS
    rpt = npad // NS
    z_cnt = jnp.zeros((cpt,), f32)
    z_acc = jnp.zeros((rpt, H), f32)
    ones_k = jnp.ones((K,), f32)

    cnt_call = pl.kernel(
        _make_count_body(cntp),
        out_type=jax.ShapeDtypeStruct((NC, cntp), f32),
        mesh=mesh,
        compiler_params=pltpu.CompilerParams(use_tc_tiling_on_sc=False),
        scratch_types=[
            pltpu.VMEM((GPC, 1, SZ), jnp.int32),
            pltpu.VMEM((K,), f32),
            pltpu.VMEM_SHARED((cntp,), f32),
        ],
    )
    cnt_p = cnt_call(skey, z_cnt, ones_k)

    def _inv_body(c_ref, o_ref):
        o_ref[...] = (1.0 / jnp.maximum(c_ref[0] + c_ref[1], 1.0))[None]

    cblk = cntp // 49
    inv_cnt = pl.pallas_call(
        _inv_body,
        grid=(49,),
        in_specs=[pl.BlockSpec((2, cblk), lambda i: (0, i))],
        out_specs=pl.BlockSpec((1, cblk), lambda i: (0, i)),
        out_shape=jax.ShapeDtypeStruct((1, cntp), f32),
    )(cnt_p).reshape(cntp)

    basisT2d = basis1.transpose(0, 2, 1).reshape(NB, H * N)
    wblk = (H * N) // 25

    def _w1_body(c_ref, b_ref, o_ref):
        o_ref[...] = jnp.dot(c_ref[...], b_ref[...],
                             preferred_element_type=f32)

    w1hn = pl.pallas_call(
        _w1_body,
        grid=(25,),
        in_specs=[pl.BlockSpec((R, NB), lambda i: (0, 0)),
                  pl.BlockSpec((NB, wblk), lambda i: (0, i))],
        out_specs=pl.BlockSpec((R, wblk), lambda i: (0, i)),
        out_shape=jax.ShapeDtypeStruct((R, H * N), f32),
    )(comp1, basisT2d)

    eye16 = jnp.eye(H, dtype=f32)

    def _tr_body(a_ref, e_ref, o_ref):
        o_ref[0] = lax.dot_general(a_ref[0], e_ref[...],
                                   (((0,), (0,)), ((), ())),
                                   preferred_element_type=f32)

    w1 = pl.pallas_call(
        _tr_body,
        grid=(R,),
        in_specs=[pl.BlockSpec((1, H, N), lambda i: (i, 0, 0)),
                  pl.BlockSpec((H, H), lambda i: (0, 0))],
        out_specs=pl.BlockSpec((1, N, H), lambda i: (i, 0, 0)),
        out_shape=jax.ShapeDtypeStruct((R, N, H), f32),
    )(w1hn.reshape(R, H, N), eye16)
    table1 = w1.reshape(R * N, H)

    def make_edge_call(rn):
        return pl.kernel(
            _make_edge_body(npad, rn, H),
            out_type=jax.ShapeDtypeStruct((NC, nacc, 128), f32),
            mesh=mesh,
            compiler_params=pltpu.CompilerParams(use_tc_tiling_on_sc=False),
            scratch_types=[
                pltpu.VMEM((2, GPC, 1, SZ), jnp.int32),
                pltpu.VMEM((2, GPC, 1, SZ), jnp.int32),
                pltpu.VMEM((2, GPC, 1, SZ), jnp.int32),
                pltpu.VMEM((K, H), f32),
                pltpu.VMEM((K,), f32),
                pltpu.VMEM((OBLK, H), f32),
                pltpu.VMEM((OBLK * H // 128, 128), f32),
                pltpu.VMEM_SHARED((npad, H), f32),
                pltpu.SemaphoreType.DMA,
                pltpu.SemaphoreType.DMA,
                pltpu.SemaphoreType.DMA,
                pltpu.SemaphoreType.DMA,
                pltpu.SemaphoreType.DMA,
                pltpu.SemaphoreType.DMA,
            ],
        )

    acc1_p = make_edge_call(R * N)(table1, gkey1, skey, dst3d, inv_cnt, z_acc)

    w2 = (comp2 @ basis2.reshape(NB, H * C)).reshape(R, H, C)
    w2s = w2.transpose(1, 0, 2).reshape(H, R * C)
    eye8 = jnp.eye(128 // H, dtype=f32)
    w2big = jnp.kron(eye8, w2s)
    r2big = jnp.kron(eye8, root2)
    rb1 = root1 + bias1[None, :]
    rb1p = jnp.concatenate([rb1, jnp.zeros((NPADT, H), f32)])
    rb128 = rb1p.reshape(nacc, 128)

    nb128 = nacc // 16

    def _x_xw_body(p_ref, rb_ref, w2_ref, x_ref, xw_ref):
        xb = jnp.maximum(p_ref[0] + p_ref[1] + rb_ref[...], 0.0)
        x_ref[...] = xb
        xw_ref[...] = jnp.dot(xb, w2_ref[...], preferred_element_type=f32)

    x128, xw = pl.pallas_call(
        _x_xw_body,
        grid=(16,),
        in_specs=[pl.BlockSpec((2, nb128, 128), lambda i: (0, i, 0)),
                  pl.BlockSpec((nb128, 128), lambda i: (i, 0)),
                  pl.BlockSpec((128, (128 // H) * R * C), lambda i: (0, 0))],
        out_specs=[pl.BlockSpec((nb128, 128), lambda i: (i, 0)),
                   pl.BlockSpec((nb128, (128 // H) * R * C),
                                lambda i: (i, 0))],
        out_shape=[jax.ShapeDtypeStruct((nacc, 128), f32),
                   jax.ShapeDtypeStruct((nacc, (128 // H) * R * C), f32)],
    )(acc1_p, rb128, w2big)
    table2 = xw.reshape(npad * R, C)

    acc2_p = make_edge_call(npad * R)(table2, gkey2, skey, dst3d, inv_cnt,
                                      z_acc)

    gsz = 128 // C
    onesb = jnp.kron(eye8, jnp.ones((C, C), f32))
    lead = jnp.kron(eye8, jnp.zeros((C, C), f32).at[0].set(1.0))
    b2t = jnp.tile(bias2, gsz)[None, :]

    def _out_body(p_ref, x_ref, r2_ref, b2_ref, ones_ref, lead_ref, o_ref):
        o = (p_ref[0] + p_ref[1] + b2_ref[...]
             + jnp.dot(x_ref[...], r2_ref[...], preferred_element_type=f32))
        m = o
        for k in (1, 2, 4, 8):
            m = jnp.maximum(m, pltpu.roll(m, 128 - k, axis=1))
        mb = jnp.dot(m, lead_ref[...], preferred_element_type=f32)
        s = o - mb
        e = jnp.exp(s)
        lse = jnp.log(jnp.dot(e, ones_ref[...], preferred_element_type=f32))
        o_ref[...] = s - lse

    out128 = pl.pallas_call(
        _out_body,
        grid=(16,),
        in_specs=[pl.BlockSpec((2, nb128, 128), lambda i: (0, i, 0)),
                  pl.BlockSpec((nb128, 128), lambda i: (i, 0)),
                  pl.BlockSpec((128, 128), lambda i: (0, 0)),
                  pl.BlockSpec((1, 128), lambda i: (0, 0)),
                  pl.BlockSpec((128, 128), lambda i: (0, 0)),
                  pl.BlockSpec((128, 128), lambda i: (0, 0))],
        out_specs=pl.BlockSpec((nb128, 128), lambda i: (i, 0)),
        out_shape=jax.ShapeDtypeStruct((nacc, 128), f32),
    )(acc2_p, x128, r2big, b2t, onesb, lead)
    return out128.reshape(npad, C)[:N]


# speedup vs baseline: 2.6694x; 1.7093x over previous
import jax
import jax.numpy as jnp
from jax import lax
from jax.experimental import pallas as pl
from jax.experimental.pallas import tpu as pltpu
from jax.experimental.pallas import tpu_sc as plsc

NC = 2
NS = 16
NW = NC * NS
SZ = 128
GPC = 16
K = SZ * GPC
NCHUNK = 25
EPW = K * NCHUNK
EPAD = NW * EPW
CPAD = 1408
NPADT = 176
OBLK = 448


def _make_count_body(cntp):
    cpt = cntp // NS

    def body(skey_hbm, zc_hbm, ones_hbm, out_hbm, sk_v, ones_v, cnt_sh):
        cid = lax.axis_index("c")
        sid = lax.axis_index("s")
        wid = cid * NS + sid
        pltpu.sync_copy(zc_hbm, cnt_sh.at[pl.ds(sid * cpt, cpt)])
        pltpu.sync_copy(ones_hbm, ones_v)
        plsc.subcore_barrier()

        def chunk(j, carry):
            row0 = wid * (NCHUNK * GPC) + j * GPC
            pltpu.sync_copy(skey_hbm.at[pl.ds(row0, GPC)], sk_v)

            def group(m, c2):
                off = pl.multiple_of(m * SZ, SZ)
                pltpu.sync_copy(ones_v.at[pl.ds(off, SZ)],
                                cnt_sh.at[sk_v.at[m, 0]], add=True)
                return c2

            lax.fori_loop(0, GPC, group, 0)
            return carry

        lax.fori_loop(0, NCHUNK, chunk, 0)
        plsc.subcore_barrier()
        pltpu.sync_copy(cnt_sh.at[pl.ds(sid * cpt, cpt)],
                        out_hbm.at[cid, pl.ds(sid * cpt, cpt)])

    return body


def _make_edge_body(npad, rn, h):
    rpt = npad // NS
    orpt = rpt * h // 128
    rph = 128 // h

    def body(table_hbm, gkey_hbm, skey_hbm, dst_hbm, inv_hbm, zr_hbm,
             out_hbm, gk_v, sk_v, d_v, rows_v, s_v, ob_v, ob128_v, acc_sh,
             gsa, gsb, ssa, ssb, isem, scsem):
        cid = lax.axis_index("c")
        sid = lax.axis_index("s")
        wid = cid * NS + sid
        pltpu.sync_copy(zr_hbm, acc_sh.at[pl.ds(sid * rpt, rpt)])
        plsc.subcore_barrier()

        def idx_issue(j, jp):
            row0 = wid * (NCHUNK * GPC) + j * GPC
            pltpu.async_copy(gkey_hbm.at[pl.ds(row0, GPC)], gk_v.at[jp],
                             isem)
            pltpu.async_copy(skey_hbm.at[pl.ds(row0, GPC)], sk_v.at[jp],
                             isem)
            pltpu.async_copy(dst_hbm.at[pl.ds(row0, GPC)], d_v.at[jp], isem)

        def idx_wait(jp):
            row0 = wid * (NCHUNK * GPC)
            pltpu.make_async_copy(gkey_hbm.at[pl.ds(row0, GPC)],
                                  gk_v.at[jp], isem).wait()
            pltpu.make_async_copy(skey_hbm.at[pl.ds(row0, GPC)],
                                  sk_v.at[jp], isem).wait()
            pltpu.make_async_copy(dst_hbm.at[pl.ds(row0, GPC)],
                                  d_v.at[jp], isem).wait()

        def gath_issue(jp, m, gs, ss):
            off = pl.multiple_of(m * SZ, SZ)
            pltpu.async_copy(table_hbm.at[gk_v.at[jp, m, 0]],
                             rows_v.at[pl.ds(off, SZ)], gs)
            pltpu.async_copy(inv_hbm.at[sk_v.at[jp, m, 0]],
                             s_v.at[pl.ds(off, SZ)], ss)

        def gath_wait(jp, m, gs, ss):
            off = pl.multiple_of(m * SZ, SZ)
            pltpu.make_async_copy(table_hbm.at[gk_v.at[jp, m, 0]],
                                  rows_v.at[pl.ds(off, SZ)], gs).wait()
            pltpu.make_async_copy(inv_hbm.at[sk_v.at[jp, m, 0]],
                                  s_v.at[pl.ds(off, SZ)], ss).wait()

        def process(jp, m):
            off = pl.multiple_of(m * SZ, SZ)

            def scale(t, c3):
                base = pl.multiple_of(off + t * 16, 16)
                sv = s_v[pl.ds(base, 16)]
                for i in range(16):
                    rows_v[base + i] = rows_v[base + i] * sv[i]
                return c3

            lax.fori_loop(0, SZ // 16, scale, 0)
            pltpu.async_copy(rows_v.at[pl.ds(off, SZ)],
                             acc_sh.at[d_v.at[jp, m, 0]], scsem, add=True)

        def sc_drain(jp):
            def one(m, c2):
                off = pl.multiple_of(m * SZ, SZ)
                pltpu.make_async_copy(rows_v.at[pl.ds(off, SZ)],
                                      acc_sh.at[d_v.at[jp, m, 0]],
                                      scsem).wait()
                return c2

            lax.fori_loop(0, GPC, one, 0)

        idx_issue(0, 0)

        def chunk(j, carry):
            jp = lax.rem(j, 2)
            idx_wait(jp)

            @pl.when(j + 1 < NCHUNK)
            def _():
                idx_issue(j + 1, 1 - jp)

            gath_issue(jp, 0, gsa, ssa)

            def pair(m2, c2):
                m = m2 * 2
                gath_issue(jp, m + 1, gsb, ssb)
                gath_wait(jp, m, gsa, ssa)
                process(jp, m)

                @pl.when(m2 + 1 < GPC // 2)
                def _():
                    gath_issue(jp, m + 2, gsa, ssa)

                gath_wait(jp, m + 1, gsb, ssb)
                process(jp, m + 1)
                return c2

            lax.fori_loop(0, GPC // 2, pair, 0)
            sc_drain(jp)
            return carry

        lax.fori_loop(0, NCHUNK, chunk, 0)
        plsc.subcore_barrier()

        def outblk(t, carry):
            pltpu.sync_copy(
                acc_sh.at[pl.ds(sid * rpt + t * OBLK, OBLK)], ob_v)

            def merge(p, c2):
                for q in range(rph):
                    ob128_v[p, pl.ds(q * h, h)] = ob_v[p * rph + q]
                return c2

            lax.fori_loop(0, OBLK // rph, merge, 0)
            pltpu.sync_copy(
                ob128_v,
                out_hbm.at[cid, pl.ds(sid * orpt + t * (OBLK // rph),
                                      OBLK // rph)])
            return carry

        lax.fori_loop(0, rpt // OBLK, outblk, 0)

    return body


def kernel(edge_index, edge_type, basis1, comp1, root1, bias1,
           basis2, comp2, root2, bias2):
    N, H = root1.shape
    R, NB = comp1.shape
    C = root2.shape[1]
    E = edge_type.shape[0]
    f32 = jnp.float32
    mesh = plsc.VectorSubcoreMesh(core_axis_name="c", subcore_axis_name="s")

    cntp = R * N + CPAD
    npad = N + NPADT
    nacc = npad * H // 128
    pade = EPAD - E

    src = edge_index[0]
    dst = edge_index[1]
    et = edge_type
    i32 = jnp.int32

    def pad3d(key, fill):
        keyp = jnp.concatenate(
            [key, jnp.full((pade,), fill, i32)])
        return keyp.reshape(EPAD // SZ, 1, SZ)

    gkey2 = pad3d(src * R + et, 0)
    skey = pad3d(et * N + dst, R * N)
    dst3d = pad3d(dst, npad - 1)

    cpt = cntp // NS
    rpt = npad // NS
    z_cnt = jnp.zeros((cpt,), f32)
    z_acc = jnp.zeros((rpt, H), f32)
    ones_k = jnp.ones((K,), f32)

    cnt_call = pl.kernel(
        _make_count_body(cntp),
        out_type=jax.ShapeDtypeStruct((NC, cntp), f32),
        mesh=mesh,
        compiler_params=pltpu.CompilerParams(use_tc_tiling_on_sc=False),
        scratch_types=[
            pltpu.VMEM((GPC, 1, SZ), jnp.int32),
            pltpu.VMEM((K,), f32),
            pltpu.VMEM_SHARED((cntp,), f32),
        ],
    )
    cnt_p = cnt_call(skey, z_cnt, ones_k)

    def _inv_body(c_ref, o_ref):
        o_ref[...] = (1.0 / jnp.maximum(c_ref[0] + c_ref[1], 1.0))[None]

    cblk = cntp // 49
    inv_cnt = pl.pallas_call(
        _inv_body,
        grid=(49,),
        in_specs=[pl.BlockSpec((2, cblk), lambda i: (0, i))],
        out_specs=pl.BlockSpec((1, cblk), lambda i: (0, i)),
        out_shape=jax.ShapeDtypeStruct((1, cntp), f32),
    )(cnt_p).reshape(cntp)

    a2d = basis1.transpose(0, 2, 1).reshape(NB * H, N)
    w1w = jnp.kron(comp1.T, jnp.eye(H, dtype=f32))
    nsteps = 12
    cb = (NB * H) // nsteps

    def _w1_body(a_ref, w_ref, o_ref):
        @pl.when(pl.program_id(0) == 0)
        def _():
            o_ref[...] = jnp.zeros_like(o_ref)

        o_ref[...] += lax.dot_general(a_ref[...], w_ref[...],
                                      (((0,), (0,)), ((), ())),
                                      preferred_element_type=f32)

    w1n128 = pl.pallas_call(
        _w1_body,
        grid=(nsteps,),
        in_specs=[pl.BlockSpec((cb, N), lambda i: (i, 0)),
                  pl.BlockSpec((cb, R * H), lambda i: (i, 0))],
        out_specs=pl.BlockSpec((N, R * H), lambda i: (0, 0)),
        out_shape=jax.ShapeDtypeStruct((N, R * H), f32),
    )(a2d, w1w)
    table1 = w1n128.reshape(N * R, H)

    def make_edge_call(rn):
        return pl.kernel(
            _make_edge_body(npad, rn, H),
            out_type=jax.ShapeDtypeStruct((NC, nacc, 128), f32),
            mesh=mesh,
            compiler_params=pltpu.CompilerParams(use_tc_tiling_on_sc=False),
            scratch_types=[
                pltpu.VMEM((2, GPC, 1, SZ), jnp.int32),
                pltpu.VMEM((2, GPC, 1, SZ), jnp.int32),
                pltpu.VMEM((2, GPC, 1, SZ), jnp.int32),
                pltpu.VMEM((K, H), f32),
                pltpu.VMEM((K,), f32),
                pltpu.VMEM((OBLK, H), f32),
                pltpu.VMEM((OBLK * H // 128, 128), f32),
                pltpu.VMEM_SHARED((npad, H), f32),
                pltpu.SemaphoreType.DMA,
                pltpu.SemaphoreType.DMA,
                pltpu.SemaphoreType.DMA,
                pltpu.SemaphoreType.DMA,
                pltpu.SemaphoreType.DMA,
                pltpu.SemaphoreType.DMA,
            ],
        )

    acc1_p = make_edge_call(R * N)(table1, gkey2, skey, dst3d, inv_cnt, z_acc)

    w2 = (comp2 @ basis2.reshape(NB, H * C)).reshape(R, H, C)
    w2s = w2.transpose(1, 0, 2).reshape(H, R * C)
    eye8 = jnp.eye(128 // H, dtype=f32)
    w2big = jnp.kron(eye8, w2s)
    r2big = jnp.kron(eye8, root2)
    rb1 = root1 + bias1[None, :]
    rb1p = jnp.concatenate([rb1, jnp.zeros((NPADT, H), f32)])
    rb128 = rb1p.reshape(nacc, 128)

    nb128 = nacc // 16

    def _x_xw_body(p_ref, rb_ref, w2_ref, x_ref, xw_ref):
        xb = jnp.maximum(p_ref[0] + p_ref[1] + rb_ref[...], 0.0)
        x_ref[...] = xb
        xw_ref[...] = jnp.dot(xb, w2_ref[...], preferred_element_type=f32)

    x128, xw = pl.pallas_call(
        _x_xw_body,
        grid=(16,),
        in_specs=[pl.BlockSpec((2, nb128, 128), lambda i: (0, i, 0)),
                  pl.BlockSpec((nb128, 128), lambda i: (i, 0)),
                  pl.BlockSpec((128, (128 // H) * R * C), lambda i: (0, 0))],
        out_specs=[pl.BlockSpec((nb128, 128), lambda i: (i, 0)),
                   pl.BlockSpec((nb128, (128 // H) * R * C),
                                lambda i: (i, 0))],
        out_shape=[jax.ShapeDtypeStruct((nacc, 128), f32),
                   jax.ShapeDtypeStruct((nacc, (128 // H) * R * C), f32)],
    )(acc1_p, rb128, w2big)
    table2 = xw.reshape(npad * R, C)

    acc2_p = make_edge_call(npad * R)(table2, gkey2, skey, dst3d, inv_cnt,
                                      z_acc)

    gsz = 128 // C
    onesb = jnp.kron(eye8, jnp.ones((C, C), f32))
    lead = jnp.kron(eye8, jnp.zeros((C, C), f32).at[0].set(1.0))
    b2t = jnp.tile(bias2, gsz)[None, :]

    def _out_body(p_ref, x_ref, r2_ref, b2_ref, ones_ref, lead_ref, o_ref):
        o = (p_ref[0] + p_ref[1] + b2_ref[...]
             + jnp.dot(x_ref[...], r2_ref[...], preferred_element_type=f32))
        m = o
        for k in (1, 2, 4, 8):
            m = jnp.maximum(m, pltpu.roll(m, 128 - k, axis=1))
        mb = jnp.dot(m, lead_ref[...], preferred_element_type=f32)
        s = o - mb
        e = jnp.exp(s)
        lse = jnp.log(jnp.dot(e, ones_ref[...], preferred_element_type=f32))
        o_ref[...] = s - lse

    out128 = pl.pallas_call(
        _out_body,
        grid=(16,),
        in_specs=[pl.BlockSpec((2, nb128, 128), lambda i: (0, i, 0)),
                  pl.BlockSpec((nb128, 128), lambda i: (i, 0)),
                  pl.BlockSpec((128, 128), lambda i: (0, 0)),
                  pl.BlockSpec((1, 128), lambda i: (0, 0)),
                  pl.BlockSpec((128, 128), lambda i: (0, 0)),
                  pl.BlockSpec((128, 128), lambda i: (0, 0))],
        out_specs=pl.BlockSpec((nb128, 128), lambda i: (i, 0)),
        out_shape=jax.ShapeDtypeStruct((nacc, 128), f32),
    )(acc2_p, x128, r2big, b2t, onesb, lead)
    return out128.reshape(npad, C)[:N]
